# baseline (speedup 1.0000x reference)
import functools

import jax
import jax.numpy as jnp
from jax import lax
from jax.experimental import pallas as pl
from jax.experimental.pallas import tpu as pltpu
from jax.experimental.pallas import tpu_sc as plsc

_BPC = 2
_PAD = 64
_NBUF = 8


@functools.lru_cache(maxsize=None)
def _make_lookup(V, D, batch, seq):
    info = plsc.get_sparse_core_info()
    NC, NS = info.num_cores, info.num_subcores
    NW = NC * NS
    assert batch % (NW * _BPC) == 0
    b_per_w = batch // NW
    n_ch = b_per_w // _BPC
    assert n_ch % _NBUF == 0
    idx_w = n_ch * _BPC * _PAD
    mesh = plsc.VectorSubcoreMesh(core_axis_name="c", subcore_axis_name="s")

    @functools.partial(
        pl.kernel,
        mesh=mesh,
        out_type=jax.ShapeDtypeStruct((batch, seq, D), jnp.float32),
        scratch_types=[
            pltpu.VMEM((idx_w,), jnp.int32),
            pltpu.VMEM((_NBUF, _BPC, seq, D), jnp.float32),
        ]
        + [pltpu.SemaphoreType.DMA] * (2 * _NBUF),
        compiler_params=pltpu.CompilerParams(use_tc_tiling_on_sc=False),
    )
    def k(idx_hbm, table_hbm, out_hbm, idx_v, rows_v, *sems):
        gsem, osem = sems[:_NBUF], sems[_NBUF:]
        wid = lax.axis_index("s") * NC + lax.axis_index("c")
        b0 = wid * b_per_w
        pltpu.sync_copy(idx_hbm.at[wid], idx_v)

        def gathers(s, b):
            return [
                pltpu.make_async_copy(
                    table_hbm.at[idx_v.at[pl.ds((s * _BPC + q) * _PAD, seq)]],
                    rows_v.at[b, q], gsem[b])
                for q in range(_BPC)
            ]

        def write(s, b):
            return pltpu.make_async_copy(
                rows_v.at[b], out_hbm.at[pl.ds(b0 + s * _BPC, _BPC)], osem[b])

        for b in range(_NBUF - 1):
            for g in gathers(b, b):
                g.start()

        def body(i, carry):
            for j in range(_NBUF):
                s = i * _NBUF + j
                bn = (j + _NBUF - 1) % _NBUF

                @pl.when(s >= 1)
                def _():
                    write(s - 1, bn).wait()

                @pl.when(s + _NBUF - 1 < n_ch)
                def _():
                    for g in gathers(s + _NBUF - 1, bn):
                        g.start()

                for g in gathers(s, j):
                    g.wait()
                write(s, j).start()
            return carry

        lax.fori_loop(0, n_ch // _NBUF, body, 0)
        write(n_ch - 1, (n_ch - 1) % _NBUF).wait()

    return k


def kernel(word_ids, table):
    batch, seq = word_ids.shape
    V, D = table.shape
    info = plsc.get_sparse_core_info()
    NW = info.num_cores * info.num_subcores
    idx = jnp.asarray(word_ids, jnp.int32).reshape(NW, batch // NW, seq)
    idx = jnp.pad(idx, ((0, 0), (0, 0), (0, _PAD - seq)))
    idx = idx.reshape(NW, (batch // NW) * _PAD)
    return _make_lookup(V, D, batch, seq)(idx, table)


# speedup vs baseline: 1.1022x; 1.1022x over previous
import functools

import jax
import jax.numpy as jnp
from jax import lax
from jax.experimental import pallas as pl
from jax.experimental.pallas import tpu as pltpu
from jax.experimental.pallas import tpu_sc as plsc

_BPC = 2
_NBUF = 8
_SPLIT = 2


@functools.lru_cache(maxsize=None)
def _make_lookup(V, D, batch, seq):
    info = plsc.get_sparse_core_info()
    NC, NS = info.num_cores, info.num_subcores
    NW = NC * NS
    assert batch % (NW * _BPC) == 0
    b_per_w = batch // NW
    n_ch = b_per_w // _BPC
    ch_rows = _BPC * seq
    lanes = 128
    assert ch_rows <= lanes
    assert n_ch % _NBUF == 0
    mesh = plsc.VectorSubcoreMesh(core_axis_name="c", subcore_axis_name="s")

    @functools.partial(
        pl.kernel,
        mesh=mesh,
        out_type=jax.ShapeDtypeStruct((batch, seq, D), jnp.float32),
        scratch_types=[
            pltpu.VMEM((n_ch, lanes), jnp.int32),
            pltpu.VMEM((_NBUF, ch_rows, D), jnp.float32),
        ]
        + [pltpu.SemaphoreType.DMA] * (2 * _NBUF),
    )
    def k(idx_hbm, table_hbm, out_hbm, idx_v, rows_v, *sems):
        gsem, osem = sems[:_NBUF], sems[_NBUF:]
        wid = lax.axis_index("s") * NC + lax.axis_index("c")
        b0 = wid * b_per_w
        pltpu.sync_copy(idx_hbm.at[wid], idx_v)

        def gather(s, b):
            return pltpu.make_async_copy(
                table_hbm.at[idx_v.at[s, pl.ds(0, ch_rows)]],
                rows_v.at[b], gsem[b])

        def writes(s, b):
            return [
                pltpu.make_async_copy(
                    rows_v.at[b, pl.ds(q * seq, seq)],
                    out_hbm.at[b0 + s * _BPC + q], osem[b])
                for q in range(_BPC)
            ]

        for b in range(_NBUF - 1):
            gather(b, b).start()

        def body(i, carry):
            for j in range(_NBUF):
                s = i * _NBUF + j
                bn = (j + _NBUF - 1) % _NBUF

                @pl.when(s >= 1)
                def _():
                    for w in writes(s - 1, bn):
                        w.wait()

                @pl.when(s + _NBUF - 1 < n_ch)
                def _():
                    gather(s + _NBUF - 1, bn).start()

                gather(s, j).wait()
                for w in writes(s, j):
                    w.start()
            return carry

        lax.fori_loop(0, n_ch // _NBUF, body, 0)
        for w in writes(n_ch - 1, (n_ch - 1) % _NBUF):
            w.wait()

    return k


def kernel(word_ids, table):
    batch, seq = word_ids.shape
    V, D = table.shape
    info = plsc.get_sparse_core_info()
    NW = info.num_cores * info.num_subcores
    bs = batch // _SPLIT
    n_ch = bs // (NW * _BPC)
    lookup = _make_lookup(V, D, bs, seq)
    parts = []
    for p in range(_SPLIT):
        ids = jnp.asarray(word_ids[p * bs:(p + 1) * bs], jnp.int32)
        idx = ids.reshape(NW, n_ch, _BPC * seq)
        idx = jnp.pad(idx, ((0, 0), (0, 0), (0, 128 - _BPC * seq)))
        parts.append(lookup(idx, table))
    return jnp.concatenate(parts, axis=0)


# speedup vs baseline: 2.9970x; 2.7191x over previous
import functools

import jax
import jax.numpy as jnp
import numpy as np
from jax import lax
from jax.experimental import pallas as pl
from jax.experimental.pallas import tpu as pltpu
from jax.experimental.pallas import tpu_sc as plsc

_CH = 128
_NBUF = 5


@functools.lru_cache(maxsize=None)
def _make_lookup(V, D, B):
    info = plsc.get_sparse_core_info()
    NC, NS = info.num_cores, info.num_subcores
    NW = NC * NS
    assert B % (NW * _CH) == 0
    b_per_w = B // NW
    n_ch = b_per_w // _CH
    assert n_ch % _NBUF == 0
    mesh = plsc.VectorSubcoreMesh(core_axis_name="c", subcore_axis_name="s")

    @functools.partial(
        pl.kernel,
        mesh=mesh,
        out_type=jax.ShapeDtypeStruct((B, D), jnp.float32),
        scratch_types=[
            pltpu.VMEM((n_ch, _CH), jnp.int32),
            pltpu.VMEM((n_ch, _CH), jnp.int32),
            pltpu.VMEM((_NBUF, _CH, D), jnp.float32),
        ]
        + [pltpu.SemaphoreType.DMA] * (2 * _NBUF),
    )
    def k(idx_hbm, oidx_hbm, table_hbm, out_hbm, idx_v, oidx_v, rows_v, *sems):
        gsem, osem = sems[:_NBUF], sems[_NBUF:]
        wid = lax.axis_index("s") * NC + lax.axis_index("c")
        pltpu.sync_copy(idx_hbm.at[wid], idx_v)
        pltpu.sync_copy(oidx_hbm.at[wid], oidx_v)

        def gather(s, b):
            return pltpu.make_async_copy(
                table_hbm.at[idx_v.at[s]], rows_v.at[b], gsem[b])

        def scatter(s, b):
            return pltpu.make_async_copy(
                rows_v.at[b], out_hbm.at[oidx_v.at[s]], osem[b])

        for b in range(_NBUF - 1):
            gather(b, b).start()

        def body(i, carry):
            for j in range(_NBUF):
                s = i * _NBUF + j
                bn = (j + _NBUF - 1) % _NBUF

                @pl.when(s >= 1)
                def _():
                    scatter(s - 1, bn).wait()

                @pl.when(s + _NBUF - 1 < n_ch)
                def _():
                    gather(s + _NBUF - 1, bn).start()

                gather(s, j).wait()
                scatter(s, j).start()
            return carry

        lax.fori_loop(0, n_ch // _NBUF, body, 0)
        scatter(n_ch - 1, (n_ch - 1) % _NBUF).wait()

    return k


def kernel(word_ids, table):
    batch, seq = word_ids.shape
    V, D = table.shape
    B = batch * seq
    info = plsc.get_sparse_core_info()
    NW = info.num_cores * info.num_subcores
    n_ch = B // (NW * _CH)
    idx = jnp.asarray(word_ids, jnp.int32).reshape(NW, n_ch, _CH)
    t = np.arange(B, dtype=np.int64)
    oidx = jnp.asarray(
        ((t % seq) * batch + t // seq).reshape(NW, n_ch, _CH), jnp.int32)
    out = _make_lookup(V, D, B)(idx, oidx, table)
    return out.reshape(seq, batch, D).transpose(1, 0, 2)


# speedup vs baseline: 3.0689x; 1.0240x over previous
import functools

import jax
import jax.numpy as jnp
from jax import lax
from jax.experimental import pallas as pl
from jax.experimental.pallas import tpu as pltpu
from jax.experimental.pallas import tpu_sc as plsc

_CH = 128
_NBUF = 5


@functools.lru_cache(maxsize=None)
def _make_lookup(V, D, B):
    info = plsc.get_sparse_core_info()
    NC, NS = info.num_cores, info.num_subcores
    NW = NC * NS
    assert B % (NW * _CH) == 0
    b_per_w = B // NW
    n_ch = b_per_w // _CH
    assert n_ch % _NBUF == 0
    mesh = plsc.VectorSubcoreMesh(core_axis_name="c", subcore_axis_name="s")

    @functools.partial(
        pl.kernel,
        mesh=mesh,
        out_type=jax.ShapeDtypeStruct((B, D), jnp.float32),
        scratch_types=[
            pltpu.VMEM((n_ch, _CH), jnp.int32),
            pltpu.VMEM((_NBUF, _CH, D), jnp.float32),
        ]
        + [pltpu.SemaphoreType.DMA] * (2 * _NBUF),
    )
    def k(idx_hbm, table_hbm, out_hbm, idx_v, rows_v, *sems):
        gsem, osem = sems[:_NBUF], sems[_NBUF:]
        wid = lax.axis_index("s") * NC + lax.axis_index("c")
        base = wid * b_per_w
        pltpu.sync_copy(idx_hbm.at[wid], idx_v)

        def gather(s, b):
            return pltpu.make_async_copy(
                table_hbm.at[idx_v.at[s]], rows_v.at[b], gsem[b])

        def write(s, b):
            return pltpu.make_async_copy(
                rows_v.at[b], out_hbm.at[pl.ds(base + s * _CH, _CH)], osem[b])

        for b in range(_NBUF - 1):
            gather(b, b).start()

        def body(i, carry):
            for j in range(_NBUF):
                s = i * _NBUF + j
                bn = (j + _NBUF - 1) % _NBUF

                @pl.when(s >= 1)
                def _():
                    write(s - 1, bn).wait()

                @pl.when(s + _NBUF - 1 < n_ch)
                def _():
                    gather(s + _NBUF - 1, bn).start()

                gather(s, j).wait()
                write(s, j).start()
            return carry

        lax.fori_loop(0, n_ch // _NBUF, body, 0)
        write(n_ch - 1, (n_ch - 1) % _NBUF).wait()

    return k


def kernel(word_ids, table):
    batch, seq = word_ids.shape
    V, D = table.shape
    B = batch * seq
    info = plsc.get_sparse_core_info()
    NW = info.num_cores * info.num_subcores
    n_ch = B // (NW * _CH)
    idx = jnp.asarray(word_ids, jnp.int32).T.reshape(NW, n_ch, _CH)
    out = _make_lookup(V, D, B)(idx, table)
    return out.reshape(seq, batch, D).transpose(1, 0, 2)


# speedup vs baseline: 3.0834x; 1.0048x over previous
import functools

import jax
import jax.numpy as jnp
from jax import lax
from jax.experimental import pallas as pl
from jax.experimental.pallas import tpu as pltpu
from jax.experimental.pallas import tpu_sc as plsc

_CH = 64
_NBUF = 10


@functools.lru_cache(maxsize=None)
def _make_lookup(V, D, B):
    info = plsc.get_sparse_core_info()
    NC, NS = info.num_cores, info.num_subcores
    NW = NC * NS
    assert B % (NW * _CH) == 0
    b_per_w = B // NW
    n_ch = b_per_w // _CH
    assert n_ch % _NBUF == 0
    mesh = plsc.VectorSubcoreMesh(core_axis_name="c", subcore_axis_name="s")

    @functools.partial(
        pl.kernel,
        mesh=mesh,
        out_type=jax.ShapeDtypeStruct((B, D), jnp.float32),
        scratch_types=[
            pltpu.VMEM((n_ch, _CH), jnp.int32),
            pltpu.VMEM((_NBUF, _CH, D), jnp.float32),
        ]
        + [pltpu.SemaphoreType.DMA] * (2 * _NBUF),
    )
    def k(idx_hbm, table_hbm, out_hbm, idx_v, rows_v, *sems):
        gsem, osem = sems[:_NBUF], sems[_NBUF:]
        wid = lax.axis_index("s") * NC + lax.axis_index("c")
        base = wid * b_per_w
        pltpu.sync_copy(idx_hbm.at[wid], idx_v)

        def gather(s, b):
            return pltpu.make_async_copy(
                table_hbm.at[idx_v.at[s]], rows_v.at[b], gsem[b])

        def write(s, b):
            return pltpu.make_async_copy(
                rows_v.at[b], out_hbm.at[pl.ds(base + s * _CH, _CH)], osem[b])

        for b in range(_NBUF - 1):
            gather(b, b).start()

        def body(i, carry):
            for j in range(_NBUF):
                s = i * _NBUF + j
                bn = (j + _NBUF - 1) % _NBUF

                @pl.when(s >= 1)
                def _():
                    write(s - 1, bn).wait()

                @pl.when(s + _NBUF - 1 < n_ch)
                def _():
                    gather(s + _NBUF - 1, bn).start()

                gather(s, j).wait()
                write(s, j).start()
            return carry

        lax.fori_loop(0, n_ch // _NBUF, body, 0)
        write(n_ch - 1, (n_ch - 1) % _NBUF).wait()

    return k


def kernel(word_ids, table):
    batch, seq = word_ids.shape
    V, D = table.shape
    B = batch * seq
    info = plsc.get_sparse_core_info()
    NW = info.num_cores * info.num_subcores
    n_ch = B // (NW * _CH)
    idx = jnp.asarray(word_ids, jnp.int32).T.reshape(NW, n_ch, _CH)
    out = _make_lookup(V, D, B)(idx, table)
    return out.reshape(seq, batch, D).transpose(1, 0, 2)
